# baseline (speedup 1.0000x reference)
import functools

import jax
import jax.numpy as jnp
from jax import lax
from jax.experimental import pallas as pl
from jax.experimental.pallas import tpu as pltpu
from jax.experimental.pallas import tpu_sc as plsc

_I32_MIN = -2147483648
_I32_MAX = 2147483647

_NC = 2
_NS = 16
_NW = _NC * _NS
_L = 16
_CH0 = 1760


def _sortable_i32(f):
    b = lax.bitcast_convert_type(f, jnp.int32)
    return jnp.where(b >= 0, b, jnp.bitwise_xor(b, _I32_MAX))


def _rowsum_t(xx):
    a = xx[:, 0:128]
    for c in range(1, 8):
        a = a + xx[:, c * 128:(c + 1) * 128]
    ta = jnp.transpose(a)
    b = ta[0:8]
    for j in range(1, 16):
        b = b + ta[j * 8:(j + 1) * 8]
    b = b[0:4] + b[4:8]
    b = b[0:2] + b[2:4]
    return b[0:1] + b[1:2]


def _dissim_select_body(x_ref, ks_ref, meta_ref, counts_ref, prev_ref, *, T, TOK, D, K):
    i = pl.program_id(0)
    row = lax.broadcasted_iota(jnp.int32, (T, TOK), 0)

    prev = prev_ref[...]
    for f in range(4):
        fr = i * 4 + f
        xb = x_ref[f]
        ss_row = _rowsum_t(xb * xb)
        ncol = jnp.transpose(jnp.sqrt(ss_row))
        xn = xb / (ncol + 1e-6)
        s = _rowsum_t(xn * prev)
        dis = jnp.where(fr == 0, jnp.float32(jnp.inf), 1.0 - s)
        ks_row = _sortable_i32(dis)
        ks_ref[...] = jnp.where(row == fr, jnp.broadcast_to(ks_row, (T, TOK)),
                                ks_ref[...])
        prev = xn
    prev_ref[...] = prev

    @pl.when(i == T // 4 - 1)
    def _select():
        ks = ks_ref[...]
        gidx = (lax.broadcasted_iota(jnp.int32, (T, TOK), 0) * TOK
                + lax.broadcasted_iota(jnp.int32, (T, TOK), 1))

        p = jnp.int32(0)
        for b in range(31, -1, -1):
            bit = jnp.int32(-2147483648) if b == 31 else jnp.int32(1 << b)
            cand = jnp.bitwise_or(p, bit)
            candc = jnp.bitwise_xor(cand, _I32_MIN)
            cnt = jnp.sum((ks >= candc).astype(jnp.int32))
            p = jnp.where(cnt >= K, cand, p)
        thr = jnp.bitwise_xor(p, _I32_MIN)

        n_gt = jnp.sum((ks > thr).astype(jnp.int32))
        need = K - n_gt

        lo = jnp.int32(0)
        hi = jnp.int32(TOK * T - 1)
        eq = ks == thr
        for _ in range(17):
            mid = lax.div(lo + hi, jnp.int32(2))
            c = jnp.sum((eq & (gidx <= mid)).astype(jnp.int32))
            ok = c >= need
            hi = jnp.where(ok, mid, hi)
            lo = jnp.where(ok, lo, mid + 1)
        mstar = lo

        mask = (ks > thr) | (eq & (gidx <= mstar))
        maskf = mask.astype(jnp.float32)

        ones_col = jnp.ones((TOK, 1), jnp.float32)
        fc = lax.dot_general(maskf, ones_col, (((1,), (0,)), ((), ())),
                             preferred_element_type=jnp.float32)
        pair = 2 * (TOK * T // _NW)
        wrow = lax.broadcasted_iota(jnp.int32, (_NW, T), 0)
        wcol = lax.broadcasted_iota(jnp.int32, (_NW, T), 1)
        fw = lax.div(wrow, 2) * (pair // TOK) + lax.rem(wrow, 2) * (_CH0 // TOK)
        W = (wcol < fw).astype(jnp.float32)
        E = (wcol == fw).astype(jnp.float32)
        part = lax.dot_general(W, maskf, (((1,), (0,)), ((), ())),
                               preferred_element_type=jnp.float32)
        frow = lax.dot_general(E, maskf, (((1,), (0,)), ((), ())),
                               preferred_element_type=jnp.float32)
        ccol = lax.broadcasted_iota(jnp.int32, (_NW, TOK), 1)
        cwb = lax.rem(lax.broadcasted_iota(jnp.int32, (_NW, TOK), 0), 2) * (_CH0 % TOK)
        part = part + frow * (ccol < cwb).astype(jnp.float32)
        offs = lax.dot_general(part, ones_col, (((1,), (0,)), ((), ())),
                               preferred_element_type=jnp.float32)

        counts_ref[...] = fc.astype(jnp.int32)
        meta = jnp.concatenate(
            [jnp.full((1, 1), thr, jnp.int32),
             jnp.full((1, 1), mstar, jnp.int32),
             offs.astype(jnp.int32),
             jnp.zeros((T - 2 - _NW, 1), jnp.int32)], axis=0)
        meta_ref[...] = meta


def _tc_stage(x, T, TOK, D, K):
    x3 = x.reshape(T, TOK, D)
    ks, meta, counts = pl.pallas_call(
        functools.partial(_dissim_select_body, T=T, TOK=TOK, D=D, K=K),
        grid=(T // 4,),
        in_specs=[pl.BlockSpec((4, TOK, D), lambda i: (i, 0, 0))],
        out_specs=[pl.BlockSpec((T, TOK), lambda i: (0, 0)),
                   pl.BlockSpec((T, 1), lambda i: (0, 0)),
                   pl.BlockSpec((T, 1), lambda i: (0, 0))],
        out_shape=[jax.ShapeDtypeStruct((T, TOK), jnp.int32),
                   jax.ShapeDtypeStruct((T, 1), jnp.int32),
                   jax.ShapeDtypeStruct((T, 1), jnp.int32)],
        scratch_shapes=[pltpu.VMEM((TOK, D), jnp.float32)],
    )(x3)
    return ks, meta, counts


def _sc_compact_gather(feat, ks_flat, meta_flat, N, D, K):
    pair = 2 * (N // _NW)
    ch1 = pair - _CH0
    chmax = max(_CH0, ch1)
    C = 32
    mesh = plsc.VectorSubcoreMesh(core_axis_name="c", subcore_axis_name="s")

    @functools.partial(
        pl.kernel, mesh=mesh,
        compiler_params=pltpu.CompilerParams(needs_layout_passes=False),
        out_type=jax.ShapeDtypeStruct((K, D), jnp.float32),
        scratch_types=[
            pltpu.VMEM((chmax,), jnp.int32),
            pltpu.VMEM((64,), jnp.int32),
            pltpu.VMEM((chmax + 2 * C,), jnp.int32),
            pltpu.VMEM((C, D), jnp.float32),
            pltpu.VMEM((C, D), jnp.float32),
            pltpu.VMEM((C, D), jnp.float32),
            pltpu.SemaphoreType.DMA,
            pltpu.SemaphoreType.DMA,
            pltpu.SemaphoreType.DMA,
            pltpu.SemaphoreType.DMA,
            pltpu.SemaphoreType.DMA,
            pltpu.SemaphoreType.DMA,
        ],
    )
    def sc_kernel(feat_hbm, ks_hbm, meta_hbm, out_hbm, ks_v, meta_v, list_v,
                  rows_v, rows2_v, rows3_v, sem, sem_g2, sem_g3, sem_s, sem_s2, sem_s3):
        cid = lax.axis_index("c")
        sid = lax.axis_index("s")
        wid = sid * _NC + cid
        base = sid * pair + cid * _CH0
        nv = lax.div(jnp.where(cid == 0, _CH0, ch1), jnp.int32(_L))
        lanes = lax.iota(jnp.int32, _L)

        cbase = jnp.minimum(base, jnp.int32(N - chmax))
        shift = base - cbase
        pltpu.sync_copy(ks_hbm.at[pl.ds(cbase, chmax)], ks_v)
        pltpu.sync_copy(meta_hbm, meta_v)

        v0 = meta_v[pl.ds(0, _L)]
        thr = jnp.sum(jnp.where(lanes == 0, v0, 0))
        mstar = jnp.sum(jnp.where(lanes == 1, v0, 0))
        j = 2 + wid
        vj = meta_v[pl.ds((j // _L) * _L, _L)]
        off = jnp.sum(jnp.where(lanes == j % _L, vj, 0))

        zero = jnp.zeros((_L,), jnp.int32)

        def prefill(q, c):
            list_v[pl.ds(q * _L, _L)] = zero
            return c

        lax.fori_loop(0, chmax // _L + 1, prefill, jnp.int32(0))

        def compact(v, cnt):
            vals = ks_v[pl.ds(shift + v * _L, _L)]
            gi = base + v * _L + lanes
            sel = (vals > thr) | ((vals == thr) & (gi <= mstar))
            pos = cnt + plsc.cumsum(sel.astype(jnp.int32)) - 1
            plsc.store_scatter(list_v, [pos], gi, mask=sel)
            return cnt + jnp.sum(sel.astype(jnp.int32))

        cnt = lax.fori_loop(0, nv, compact, jnp.int32(0))
        nchunks = lax.div(cnt + (C - 1), jnp.int32(C))

        cnt1 = jnp.maximum(cnt - 1, 0)
        lv = list_v[pl.ds(lax.div(cnt1, jnp.int32(_L)) * _L, _L)]
        last_gi = jnp.sum(jnp.where(lanes == lax.rem(cnt1, jnp.int32(_L)), lv, 0))

        vbase = lax.div(cnt, jnp.int32(_L))

        def patch(q, c):
            start = (vbase + q) * _L
            pos = start + lanes
            old = list_v[pl.ds(start, _L)]
            list_v[pl.ds(start, _L)] = jnp.where(pos < cnt, old, last_gi)
            return c

        lax.fori_loop(0, (C // _L) + 1, patch, jnp.int32(0))

        def dst_reg(i, q):
            pos = i * C + q * _L + lanes
            return off + jnp.minimum(pos, cnt1)

        bufs = (rows_v, rows2_v, rows3_v)
        gsems = (sem, sem_g2, sem_g3)
        ssems = (sem_s, sem_s2, sem_s3)
        nsub = C // _L
        NB = 3

        def drain_scatter(j, b):
            for q in range(nsub):
                pltpu.make_async_copy(bufs[b].at[pl.ds(q * _L, _L)],
                                      out_hbm.at[dst_reg(j, q)], ssems[b]).wait()

        @pl.when(nchunks > 0)
        def _go():
            pltpu.async_copy(feat_hbm.at[list_v.at[pl.ds(0, C)]], bufs[0], gsems[0])

            def step(r3, c):
                for sub in range(NB):
                    i = r3 * NB + sub
                    nxt = (sub + 1) % NB

                    @pl.when(i < nchunks)
                    def _body():
                        @pl.when(i >= 2)
                        def _drain_prev():
                            drain_scatter(i - 2, nxt)

                        pltpu.make_async_copy(
                            feat_hbm.at[list_v.at[pl.ds(i * C, C)]], bufs[sub],
                            gsems[sub]).wait()
                        for q in range(nsub):
                            pltpu.async_copy(bufs[sub].at[pl.ds(q * _L, _L)],
                                             out_hbm.at[dst_reg(i, q)], ssems[sub])

                        @pl.when(i + 1 < nchunks)
                        def _prefetch():
                            pltpu.async_copy(
                                feat_hbm.at[list_v.at[pl.ds((i + 1) * C, C)]],
                                bufs[nxt], gsems[nxt])

                return c

            lax.fori_loop(0, lax.div(nchunks + (NB - 1), jnp.int32(NB)), step,
                          jnp.int32(0))

            for back in (1, 0):
                j = nchunks - 1 - back

                @pl.when(j >= jnp.int32(0))
                def _drain_tail(j=j):
                    for b in range(NB):
                        @pl.when(lax.rem(j, jnp.int32(NB)) == b)
                        def _d(b=b):
                            drain_scatter(j, b)

    return sc_kernel(feat, ks_flat, meta_flat)


def kernel(videos_features, t, h, w):
    N, D = videos_features.shape
    TOK = 32 * 32
    T = N // TOK
    K = (N + 1) // 2

    ks, meta, counts = _tc_stage(videos_features, T, TOK, D, K)
    ks_flat = ks.reshape(N)
    meta_flat = meta.reshape(T)[:64]
    preserved = _sc_compact_gather(videos_features, ks_flat, meta_flat, N, D, K)
    return preserved, counts.reshape(T).astype(jnp.int32)


# speedup vs baseline: 1.0083x; 1.0083x over previous
import functools

import jax
import jax.numpy as jnp
from jax import lax
from jax.experimental import pallas as pl
from jax.experimental.pallas import tpu as pltpu
from jax.experimental.pallas import tpu_sc as plsc

_I32_MIN = -2147483648
_I32_MAX = 2147483647

_NC = 2
_NS = 16
_NW = _NC * _NS
_L = 16
_CH0 = 1792


def _sortable_i32(f):
    b = lax.bitcast_convert_type(f, jnp.int32)
    return jnp.where(b >= 0, b, jnp.bitwise_xor(b, _I32_MAX))


def _rowsum_t(xx):
    a = xx[:, 0:128]
    for c in range(1, 8):
        a = a + xx[:, c * 128:(c + 1) * 128]
    ta = jnp.transpose(a)
    b = ta[0:8]
    for j in range(1, 16):
        b = b + ta[j * 8:(j + 1) * 8]
    b = b[0:4] + b[4:8]
    b = b[0:2] + b[2:4]
    return b[0:1] + b[1:2]


def _dissim_select_body(x_ref, ks_ref, meta_ref, counts_ref, prev_ref, *, T, TOK, D, K):
    i = pl.program_id(0)
    row = lax.broadcasted_iota(jnp.int32, (T, TOK), 0)

    prev = prev_ref[...]
    for f in range(4):
        fr = i * 4 + f
        xb = x_ref[f]
        ss_row = _rowsum_t(xb * xb)
        ncol = jnp.transpose(jnp.sqrt(ss_row))
        xn = xb / (ncol + 1e-6)
        s = _rowsum_t(xn * prev)
        dis = jnp.where(fr == 0, jnp.float32(jnp.inf), 1.0 - s)
        ks_row = _sortable_i32(dis)
        ks_ref[...] = jnp.where(row == fr, jnp.broadcast_to(ks_row, (T, TOK)),
                                ks_ref[...])
        prev = xn
    prev_ref[...] = prev

    @pl.when(i == T // 4 - 1)
    def _select():
        ks = ks_ref[...]
        gidx = (lax.broadcasted_iota(jnp.int32, (T, TOK), 0) * TOK
                + lax.broadcasted_iota(jnp.int32, (T, TOK), 1))

        p = jnp.int32(0)
        for b in range(31, -1, -1):
            bit = jnp.int32(-2147483648) if b == 31 else jnp.int32(1 << b)
            cand = jnp.bitwise_or(p, bit)
            candc = jnp.bitwise_xor(cand, _I32_MIN)
            cnt = jnp.sum((ks >= candc).astype(jnp.int32))
            p = jnp.where(cnt >= K, cand, p)
        thr = jnp.bitwise_xor(p, _I32_MIN)

        n_gt = jnp.sum((ks > thr).astype(jnp.int32))
        need = K - n_gt

        lo = jnp.int32(0)
        hi = jnp.int32(TOK * T - 1)
        eq = ks == thr
        for _ in range(17):
            mid = lax.div(lo + hi, jnp.int32(2))
            c = jnp.sum((eq & (gidx <= mid)).astype(jnp.int32))
            ok = c >= need
            hi = jnp.where(ok, mid, hi)
            lo = jnp.where(ok, lo, mid + 1)
        mstar = lo

        mask = (ks > thr) | (eq & (gidx <= mstar))
        maskf = mask.astype(jnp.float32)

        ones_col = jnp.ones((TOK, 1), jnp.float32)
        fc = lax.dot_general(maskf, ones_col, (((1,), (0,)), ((), ())),
                             preferred_element_type=jnp.float32)
        pair = 2 * (TOK * T // _NW)
        wrow = lax.broadcasted_iota(jnp.int32, (_NW, T), 0)
        wcol = lax.broadcasted_iota(jnp.int32, (_NW, T), 1)
        fw = lax.div(wrow, 2) * (pair // TOK) + lax.rem(wrow, 2) * (_CH0 // TOK)
        W = (wcol < fw).astype(jnp.float32)
        E = (wcol == fw).astype(jnp.float32)
        part = lax.dot_general(W, maskf, (((1,), (0,)), ((), ())),
                               preferred_element_type=jnp.float32)
        frow = lax.dot_general(E, maskf, (((1,), (0,)), ((), ())),
                               preferred_element_type=jnp.float32)
        ccol = lax.broadcasted_iota(jnp.int32, (_NW, TOK), 1)
        cwb = lax.rem(lax.broadcasted_iota(jnp.int32, (_NW, TOK), 0), 2) * (_CH0 % TOK)
        part = part + frow * (ccol < cwb).astype(jnp.float32)
        offs = lax.dot_general(part, ones_col, (((1,), (0,)), ((), ())),
                               preferred_element_type=jnp.float32)

        counts_ref[...] = fc.astype(jnp.int32)
        meta = jnp.concatenate(
            [jnp.full((1, 1), thr, jnp.int32),
             jnp.full((1, 1), mstar, jnp.int32),
             offs.astype(jnp.int32),
             jnp.zeros((T - 2 - _NW, 1), jnp.int32)], axis=0)
        meta_ref[...] = meta


def _tc_stage(x, T, TOK, D, K):
    x3 = x.reshape(T, TOK, D)
    ks, meta, counts = pl.pallas_call(
        functools.partial(_dissim_select_body, T=T, TOK=TOK, D=D, K=K),
        grid=(T // 4,),
        in_specs=[pl.BlockSpec((4, TOK, D), lambda i: (i, 0, 0))],
        out_specs=[pl.BlockSpec((T, TOK), lambda i: (0, 0)),
                   pl.BlockSpec((T, 1), lambda i: (0, 0)),
                   pl.BlockSpec((T, 1), lambda i: (0, 0))],
        out_shape=[jax.ShapeDtypeStruct((T, TOK), jnp.int32),
                   jax.ShapeDtypeStruct((T, 1), jnp.int32),
                   jax.ShapeDtypeStruct((T, 1), jnp.int32)],
        scratch_shapes=[pltpu.VMEM((TOK, D), jnp.float32)],
    )(x3)
    return ks, meta, counts


def _sc_compact_gather(feat, ks_flat, meta_flat, N, D, K):
    pair = 2 * (N // _NW)
    ch1 = pair - _CH0
    chmax = max(_CH0, ch1)
    C = 32
    mesh = plsc.VectorSubcoreMesh(core_axis_name="c", subcore_axis_name="s")

    @functools.partial(
        pl.kernel, mesh=mesh,
        compiler_params=pltpu.CompilerParams(needs_layout_passes=False),
        out_type=jax.ShapeDtypeStruct((K, D), jnp.float32),
        scratch_types=[
            pltpu.VMEM((chmax,), jnp.int32),
            pltpu.VMEM((64,), jnp.int32),
            pltpu.VMEM((chmax + 2 * C,), jnp.int32),
            pltpu.VMEM((C, D), jnp.float32),
            pltpu.VMEM((C, D), jnp.float32),
            pltpu.VMEM((C, D), jnp.float32),
            pltpu.SemaphoreType.DMA,
            pltpu.SemaphoreType.DMA,
            pltpu.SemaphoreType.DMA,
            pltpu.SemaphoreType.DMA,
            pltpu.SemaphoreType.DMA,
            pltpu.SemaphoreType.DMA,
        ],
    )
    def sc_kernel(feat_hbm, ks_hbm, meta_hbm, out_hbm, ks_v, meta_v, list_v,
                  rows_v, rows2_v, rows3_v, sem, sem_g2, sem_g3, sem_s, sem_s2, sem_s3):
        cid = lax.axis_index("c")
        sid = lax.axis_index("s")
        wid = sid * _NC + cid
        base = sid * pair + cid * _CH0
        nv = lax.div(jnp.where(cid == 0, _CH0, ch1), jnp.int32(_L))
        lanes = lax.iota(jnp.int32, _L)

        cbase = jnp.minimum(base, jnp.int32(N - chmax))
        shift = base - cbase
        pltpu.sync_copy(ks_hbm.at[pl.ds(cbase, chmax)], ks_v)
        pltpu.sync_copy(meta_hbm, meta_v)

        v0 = meta_v[pl.ds(0, _L)]
        thr = jnp.sum(jnp.where(lanes == 0, v0, 0))
        mstar = jnp.sum(jnp.where(lanes == 1, v0, 0))
        j = 2 + wid
        vj = meta_v[pl.ds((j // _L) * _L, _L)]
        off = jnp.sum(jnp.where(lanes == j % _L, vj, 0))

        zero = jnp.zeros((_L,), jnp.int32)

        def prefill(q, c):
            list_v[pl.ds(q * _L, _L)] = zero
            return c

        lax.fori_loop(0, chmax // _L + 1, prefill, jnp.int32(0))

        def compact(v, cnt):
            vals = ks_v[pl.ds(shift + v * _L, _L)]
            gi = base + v * _L + lanes
            sel = (vals > thr) | ((vals == thr) & (gi <= mstar))
            pos = cnt + plsc.cumsum(sel.astype(jnp.int32)) - 1
            plsc.store_scatter(list_v, [pos], gi, mask=sel)
            return cnt + jnp.sum(sel.astype(jnp.int32))

        cnt = lax.fori_loop(0, nv, compact, jnp.int32(0))
        nchunks = lax.div(cnt + (C - 1), jnp.int32(C))

        cnt1 = jnp.maximum(cnt - 1, 0)
        lv = list_v[pl.ds(lax.div(cnt1, jnp.int32(_L)) * _L, _L)]
        last_gi = jnp.sum(jnp.where(lanes == lax.rem(cnt1, jnp.int32(_L)), lv, 0))

        vbase = lax.div(cnt, jnp.int32(_L))

        def patch(q, c):
            start = (vbase + q) * _L
            pos = start + lanes
            old = list_v[pl.ds(start, _L)]
            list_v[pl.ds(start, _L)] = jnp.where(pos < cnt, old, last_gi)
            return c

        lax.fori_loop(0, (C // _L) + 1, patch, jnp.int32(0))

        def dst_reg(i, q):
            pos = i * C + q * _L + lanes
            return off + jnp.minimum(pos, cnt1)

        bufs = (rows_v, rows2_v, rows3_v)
        gsems = (sem, sem_g2, sem_g3)
        ssems = (sem_s, sem_s2, sem_s3)
        nsub = C // _L
        NB = 3

        def drain_scatter(j, b):
            for q in range(nsub):
                pltpu.make_async_copy(bufs[b].at[pl.ds(q * _L, _L)],
                                      out_hbm.at[dst_reg(j, q)], ssems[b]).wait()

        @pl.when(nchunks > 0)
        def _go():
            pltpu.async_copy(feat_hbm.at[list_v.at[pl.ds(0, C)]], bufs[0], gsems[0])

            def step(r3, c):
                for sub in range(NB):
                    i = r3 * NB + sub
                    nxt = (sub + 1) % NB

                    @pl.when(i < nchunks)
                    def _body():
                        @pl.when(i >= 2)
                        def _drain_prev():
                            drain_scatter(i - 2, nxt)

                        pltpu.make_async_copy(
                            feat_hbm.at[list_v.at[pl.ds(i * C, C)]], bufs[sub],
                            gsems[sub]).wait()
                        for q in range(nsub):
                            pltpu.async_copy(bufs[sub].at[pl.ds(q * _L, _L)],
                                             out_hbm.at[dst_reg(i, q)], ssems[sub])

                        @pl.when(i + 1 < nchunks)
                        def _prefetch():
                            pltpu.async_copy(
                                feat_hbm.at[list_v.at[pl.ds((i + 1) * C, C)]],
                                bufs[nxt], gsems[nxt])

                return c

            lax.fori_loop(0, lax.div(nchunks + (NB - 1), jnp.int32(NB)), step,
                          jnp.int32(0))

            for back in (1, 0):
                j = nchunks - 1 - back

                @pl.when(j >= jnp.int32(0))
                def _drain_tail(j=j):
                    for b in range(NB):
                        @pl.when(lax.rem(j, jnp.int32(NB)) == b)
                        def _d(b=b):
                            drain_scatter(j, b)

    return sc_kernel(feat, ks_flat, meta_flat)


def kernel(videos_features, t, h, w):
    N, D = videos_features.shape
    TOK = 32 * 32
    T = N // TOK
    K = (N + 1) // 2

    ks, meta, counts = _tc_stage(videos_features, T, TOK, D, K)
    ks_flat = ks.reshape(N)
    meta_flat = meta.reshape(T)[:64]
    preserved = _sc_compact_gather(videos_features, ks_flat, meta_flat, N, D, K)
    return preserved, counts.reshape(T).astype(jnp.int32)
